# baseline (speedup 1.0000x reference)
import functools

import jax
import jax.numpy as jnp
from jax import lax
from jax.experimental import pallas as pl
from jax.experimental.pallas import tpu as pltpu, tpu_sc as plsc

HIDDEN = 2048
NUM_EXPERTS = 16
TOP_K = 2
N_TOKENS = 4 * 4096

_NC, _NS = 2, 16
_NW = _NC * _NS
_ROWS_PER_W = N_TOKENS // _NW
_TILES = _ROWS_PER_W // 16


def _matmul_body(x_ref, wt_ref, logits_ref):
    logits_ref[...] = jax.lax.dot_general(
        x_ref[...], wt_ref[...], (((1,), (0,)), ((), ())),
        preferred_element_type=jnp.float32)


def _routing_body(logits_hbm, probs_hbm, w_hbm, i_hbm, lv, pv, wv, iv):
    wid = lax.axis_index("c") * _NS + lax.axis_index("s")
    base = wid * _ROWS_PER_W
    nwords = _ROWS_PER_W * NUM_EXPERTS
    pltpu.sync_copy(logits_hbm.at[pl.ds(base * NUM_EXPERTS, nwords)], lv)

    def tile_body(t, _):
        iota = lax.iota(jnp.int32, 16)
        fidx0 = t * (16 * NUM_EXPERTS) + iota * NUM_EXPERTS
        regs = [plsc.load_gather(lv, [fidx0 + e]) for e in range(NUM_EXPERTS)]
        m = regs[0]
        for e in range(1, NUM_EXPERTS):
            m = jnp.maximum(m, regs[e])
        es = [jnp.exp(r - m) for r in regs]
        s = es[0]
        for e in range(1, NUM_EXPERTS):
            s = s + es[e]
        inv = 1.0 / s
        ps = [ee * inv for ee in es]
        for e in range(NUM_EXPERTS):
            plsc.store_scatter(pv, [fidx0 + e], ps[e])
        m1 = ps[0]
        i1 = jnp.zeros((16,), jnp.int32)
        m2 = jnp.full((16,), -1.0, jnp.float32)
        i2 = jnp.zeros((16,), jnp.int32)
        for e in range(1, NUM_EXPERTS):
            ev = jnp.full((16,), e, jnp.int32)
            gt1 = ps[e] > m1
            gt2 = ps[e] > m2
            m2 = jnp.where(gt1, m1, jnp.where(gt2, ps[e], m2))
            i2 = jnp.where(gt1, i1, jnp.where(gt2, ev, i2))
            m1 = jnp.where(gt1, ps[e], m1)
            i1 = jnp.where(gt1, ev, i1)
        denom = m1 + m2
        li2 = t * (16 * TOP_K) + iota * TOP_K
        li2p = li2 + 1
        plsc.store_scatter(wv, [li2], m1 / denom)
        plsc.store_scatter(wv, [li2p], m2 / denom)
        plsc.store_scatter(iv, [li2], i1)
        plsc.store_scatter(iv, [li2p], i2)
        return 0

    lax.fori_loop(0, _TILES, tile_body, 0)
    pltpu.sync_copy(pv, probs_hbm.at[pl.ds(base * NUM_EXPERTS, nwords)])
    pltpu.sync_copy(wv, w_hbm.at[pl.ds(TOP_K * base, TOP_K * _ROWS_PER_W)])
    pltpu.sync_copy(iv, i_hbm.at[pl.ds(TOP_K * base, TOP_K * _ROWS_PER_W)])


_routing = pl.kernel(
    _routing_body,
    out_type=[
        jax.ShapeDtypeStruct((N_TOKENS * NUM_EXPERTS,), jnp.float32),
        jax.ShapeDtypeStruct((TOP_K * N_TOKENS,), jnp.float32),
        jax.ShapeDtypeStruct((TOP_K * N_TOKENS,), jnp.int32),
    ],
    mesh=plsc.VectorSubcoreMesh(core_axis_name="c", subcore_axis_name="s",
                                num_cores=_NC, num_subcores=_NS),
    scratch_types=[
        pltpu.VMEM((_ROWS_PER_W * NUM_EXPERTS,), jnp.float32),
        pltpu.VMEM((_ROWS_PER_W * NUM_EXPERTS,), jnp.float32),
        pltpu.VMEM((TOP_K * _ROWS_PER_W,), jnp.float32),
        pltpu.VMEM((TOP_K * _ROWS_PER_W,), jnp.int32),
    ],
    compiler_params=pltpu.CompilerParams(needs_layout_passes=False),
)


@jax.jit
def kernel(x, W):
    B, S, H = x.shape
    N = B * S
    x2 = x.reshape(N, H)
    wt = W.T

    block_rows = 1024
    logits = pl.pallas_call(
        _matmul_body,
        grid=(N // block_rows,),
        in_specs=[
            pl.BlockSpec((block_rows, H), lambda i: (i, 0)),
            pl.BlockSpec((H, NUM_EXPERTS), lambda i: (0, 0)),
        ],
        out_specs=pl.BlockSpec((block_rows, NUM_EXPERTS), lambda i: (i, 0)),
        out_shape=jax.ShapeDtypeStruct((N, NUM_EXPERTS), jnp.float32),
    )(x2, wt)

    probs_flat, w_flat, i_flat = _routing(logits.reshape(-1))
    probs = probs_flat.reshape(N, NUM_EXPERTS)
    routing_weights = w_flat.reshape(B, S, TOP_K)
    expert_indices = i_flat.reshape(B, S, TOP_K)
    return (routing_weights, expert_indices, logits, probs)


# speedup vs baseline: 1.4270x; 1.4270x over previous
import functools

import jax
import jax.numpy as jnp
from jax.experimental import pallas as pl

HIDDEN = 2048
NUM_EXPERTS = 16
TOP_K = 2


def _router_kernel(x_ref, wt_ref, logits_ref, probs_ref, weights_ref, idx_ref):
    x_blk = x_ref[...]
    wt = wt_ref[...]
    logits = jax.lax.dot_general(
        x_blk, wt, (((1,), (0,)), ((), ())),
        preferred_element_type=jnp.float32)
    logits_ref[...] = logits

    m = jnp.max(logits, axis=-1, keepdims=True)
    e = jnp.exp(logits - m)
    s = jnp.sum(e, axis=-1, keepdims=True)
    probs = e / s
    probs_ref[...] = probs

    iota = jax.lax.broadcasted_iota(jnp.int32, probs.shape, 1)
    p1 = jnp.max(probs, axis=-1, keepdims=True)
    i1 = jnp.argmax(probs, axis=-1, keepdims=True).astype(jnp.int32)
    masked = jnp.where(iota == i1, -jnp.inf, probs)
    p2 = jnp.max(masked, axis=-1, keepdims=True)
    i2 = jnp.argmax(masked, axis=-1, keepdims=True).astype(jnp.int32)
    denom = p1 + p2
    weights_ref[...] = jnp.concatenate([p1 / denom, p2 / denom], axis=-1)
    idx_ref[...] = jnp.concatenate([i1, i2], axis=-1)


@jax.jit
def kernel(x, W):
    B, S, H = x.shape
    N = B * S
    x2 = x.reshape(N, H)
    wt = W.T

    block_rows = 1024
    grid = (N // block_rows,)

    logits, probs, weights, idx = pl.pallas_call(
        _router_kernel,
        grid=grid,
        in_specs=[
            pl.BlockSpec((block_rows, H), lambda i: (i, 0)),
            pl.BlockSpec((H, NUM_EXPERTS), lambda i: (0, 0)),
        ],
        out_specs=[
            pl.BlockSpec((block_rows, NUM_EXPERTS), lambda i: (i, 0)),
            pl.BlockSpec((block_rows, NUM_EXPERTS), lambda i: (i, 0)),
            pl.BlockSpec((block_rows, TOP_K), lambda i: (i, 0)),
            pl.BlockSpec((block_rows, TOP_K), lambda i: (i, 0)),
        ],
        out_shape=[
            jax.ShapeDtypeStruct((N, NUM_EXPERTS), jnp.float32),
            jax.ShapeDtypeStruct((N, NUM_EXPERTS), jnp.float32),
            jax.ShapeDtypeStruct((N, TOP_K), jnp.float32),
            jax.ShapeDtypeStruct((N, TOP_K), jnp.int32),
        ],
    )(x2, wt)

    routing_weights = weights.reshape(B, S, TOP_K)
    expert_indices = idx.reshape(B, S, TOP_K)
    return (routing_weights, expert_indices, logits, probs)
